# baseline (speedup 1.0000x reference)
import functools

import jax
import jax.numpy as jnp
from jax import lax
from jax.experimental import pallas as pl
from jax.experimental.pallas import tpu as pltpu
from jax.experimental.pallas import tpu_sc as plsc

N_FACTORS = 100
BATCH = 16384
LANES = 16
NC = 2
NS = 16
NW = NC * NS
B_PER_W = BATCH // NW
CH = 128
NCH = B_PER_W // CH
CGP = CH // LANES
NSLOTS = 2
MV_NB = 32768


def _mv_body(x_ref, w_ref, o_ref):
    o_ref[...] = jnp.dot(w_ref[...], x_ref[...],
                         preferred_element_type=jnp.float32)[0]


def _tc_project(xt, w_row, n_out):
    f, n = xt.shape
    grid = (pl.cdiv(n_out, MV_NB),)
    return pl.pallas_call(
        _mv_body,
        grid=grid,
        in_specs=[
            pl.BlockSpec((f, MV_NB), lambda i: (0, i)),
            pl.BlockSpec((1, f), lambda i: (0, 0)),
        ],
        out_specs=pl.BlockSpec((MV_NB,), lambda i: (i,)),
        out_shape=jax.ShapeDtypeStruct((n_out,), jnp.float32),
    )(xt, w_row)


def _sc_body(uid_hbm, cid_hbm, pu_hbm, pc_hbm, bb_hbm,
             out_hbm,
             uidx_v, cidx_v, urow_v, crow_v, ub0, ub1, cb0, cb1,
             bb_v, rot_v, out_v,
             su0, su1, sc0, sc1):
    ubufs = (ub0, ub1)
    cbufs = (cb0, cb1)
    usems = (su0, su1)
    csems = (sc0, sc1)

    wid = lax.axis_index("s") * NC + lax.axis_index("c")
    base = wid * B_PER_W

    pltpu.sync_copy(uid_hbm.at[pl.ds(base, B_PER_W)], uidx_v)
    pltpu.sync_copy(cid_hbm.at[pl.ds(base, B_PER_W)], cidx_v)
    pltpu.sync_copy(bb_hbm, bb_v)

    for q in range(B_PER_W // LANES):
        off = q * LANES
        urow_v[pl.ds(off, LANES)] = lax.shift_right_logical(
            uidx_v[pl.ds(off, LANES)], 7)
        crow_v[pl.ds(off, LANES)] = lax.shift_right_logical(
            cidx_v[pl.ds(off, LANES)], 7)

    def start(k, slot):
        hu = pltpu.async_copy(pu_hbm.at[urow_v.at[pl.ds(k * CH, CH)]],
                              ubufs[slot], usems[slot])
        hc = pltpu.async_copy(pc_hbm.at[crow_v.at[pl.ds(k * CH, CH)]],
                              cbufs[slot], csems[slot])
        return hu, hc

    bvec = bb_v[:]
    lane = lax.iota(jnp.int32, LANES)

    handles = [None] * NCH
    for k in range(NSLOTS):
        handles[k] = start(k, k % NSLOTS)

    for k in range(NCH):
        slot = k % NSLOTS
        hu, hc = handles[k]
        hu.wait()
        hc.wait()
        ubuf = ubufs[slot]
        cbuf = cbufs[slot]

        def gbody(g, _, ubuf=ubuf, cbuf=cbuf, k=k):
            goff = pl.multiple_of(k * CH + g * LANES, LANES)
            iu = uidx_v[pl.ds(goff, LANES)]
            ic = cidx_v[pl.ds(goff, LANES)]
            res = bvec
            for j in range(LANES):
                r = g * LANES + j

                def pick(buf, idx_vec, rb):
                    w = idx_vec[j] & 127
                    coff = pl.multiple_of(w & 112, LANES)
                    v = buf[r, pl.ds(coff, LANES)]
                    rot_v[pl.ds(rb, LANES)] = v
                    rot_v[pl.ds(rb + LANES, LANES)] = v
                    return rot_v[pl.ds(rb + (((w & 15) - j + LANES) & 15),
                                       LANES)]

                tu = pick(ubuf, iu, 4 * LANES * j)
                tc_ = pick(cbuf, ic, 4 * LANES * j + 2 * LANES)
                res = jnp.where(lane == j, res + tu + tc_, res)
            out_v[pl.ds(goff, LANES)] = res
            return 0

        lax.fori_loop(0, CGP, gbody, 0)

        nxt = k + NSLOTS
        if nxt < NCH:
            handles[nxt] = start(nxt, slot)

    pltpu.sync_copy(out_v, out_hbm.at[pl.ds(base, B_PER_W)])


def _make_sc_gather(nru, nrc):
    return functools.partial(
        pl.kernel,
        mesh=plsc.VectorSubcoreMesh(core_axis_name="c", subcore_axis_name="s"),
        out_type=jax.ShapeDtypeStruct((BATCH,), jnp.float32),
        scratch_types=[
            pltpu.VMEM((B_PER_W,), jnp.int32),
            pltpu.VMEM((B_PER_W,), jnp.int32),
            pltpu.VMEM((B_PER_W,), jnp.int32),
            pltpu.VMEM((B_PER_W,), jnp.int32),
            pltpu.VMEM((CH, 128), jnp.float32),
            pltpu.VMEM((CH, 128), jnp.float32),
            pltpu.VMEM((CH, 128), jnp.float32),
            pltpu.VMEM((CH, 128), jnp.float32),
            pltpu.VMEM((LANES,), jnp.float32),
            pltpu.VMEM((4 * LANES * LANES,), jnp.float32),
            pltpu.VMEM((B_PER_W,), jnp.float32),
            pltpu.SemaphoreType.DMA,
            pltpu.SemaphoreType.DMA,
            pltpu.SemaphoreType.DMA,
            pltpu.SemaphoreType.DMA,
        ],
    )(_sc_body)


_SC_GATHER = None


def kernel(user_ids, course_ids, user_factors, course_factors, fc_w, fc_b):
    global _SC_GATHER
    nu = user_factors.shape[0]
    ncr = course_factors.shape[0]

    wu_row = fc_w[:N_FACTORS].T
    wc_row = fc_w[N_FACTORS:].T
    nru = -(-nu // 1024) * 8
    nrc = -(-ncr // 1024) * 8
    p_u = _tc_project(user_factors.T, wu_row, nru * 128)
    p_c = _tc_project(course_factors.T, wc_row, nrc * 128)
    pu2 = p_u.reshape(nru, 128)
    pc2 = p_c.reshape(nrc, 128)
    bb16 = jnp.broadcast_to(fc_b, (LANES,)).astype(jnp.float32)

    if _SC_GATHER is None:
        _SC_GATHER = _make_sc_gather(nru, nrc)
    return _SC_GATHER(user_ids.astype(jnp.int32),
                      course_ids.astype(jnp.int32),
                      pu2, pc2, bb16)


# speedup vs baseline: 1.0175x; 1.0175x over previous
import functools

import jax
import jax.numpy as jnp
from jax import lax
from jax.experimental import pallas as pl
from jax.experimental.pallas import tpu as pltpu
from jax.experimental.pallas import tpu_sc as plsc

N_FACTORS = 100
BATCH = 16384
LANES = 16
NC = 2
NS = 16
NW = NC * NS
B_PER_W = BATCH // NW
CH = 128
NCH = B_PER_W // CH
CGP = CH // LANES
NSLOTS = 2
MV_NB = 16384


def _mv_body(x_ref, w_ref, o_ref):
    o_ref[...] = jnp.dot(w_ref[...], x_ref[...],
                         preferred_element_type=jnp.float32)[0]


def _tc_project(xt, w_row, n_out):
    f, n = xt.shape
    grid = (pl.cdiv(n_out, MV_NB),)
    return pl.pallas_call(
        _mv_body,
        grid=grid,
        in_specs=[
            pl.BlockSpec((f, MV_NB), lambda i: (0, i)),
            pl.BlockSpec((1, f), lambda i: (0, 0)),
        ],
        out_specs=pl.BlockSpec((MV_NB,), lambda i: (i,)),
        out_shape=jax.ShapeDtypeStruct((n_out,), jnp.float32),
    )(xt, w_row)


def _sc_gather_body(idx_hbm, p_hbm, base_hbm,
                    out_hbm,
                    idx_v, row_v, b0, b1, base_v, rot_v, out_v,
                    s0, s1):
    bufs = (b0, b1)
    sems = (s0, s1)

    wid = lax.axis_index("s") * NC + lax.axis_index("c")
    base = wid * B_PER_W

    pltpu.sync_copy(idx_hbm.at[pl.ds(base, B_PER_W)], idx_v)
    pltpu.sync_copy(base_hbm.at[pl.ds(base, B_PER_W)], base_v)

    for q in range(B_PER_W // LANES):
        off = q * LANES
        row_v[pl.ds(off, LANES)] = lax.shift_right_logical(
            idx_v[pl.ds(off, LANES)], 7)

    def start(k, slot):
        return pltpu.async_copy(p_hbm.at[row_v.at[pl.ds(k * CH, CH)]],
                                bufs[slot], sems[slot])

    lane = lax.iota(jnp.int32, LANES)

    handles = [None] * NCH
    for k in range(NSLOTS):
        handles[k] = start(k, k % NSLOTS)

    for k in range(NCH):
        slot = k % NSLOTS
        handles[k].wait()
        buf = bufs[slot]

        def gbody(g, _, buf=buf, k=k):
            goff = pl.multiple_of(k * CH + g * LANES, LANES)
            iv = idx_v[pl.ds(goff, LANES)]
            res = base_v[pl.ds(goff, LANES)]
            for j in range(LANES):
                r = g * LANES + j
                w = iv[j] & 127
                coff = pl.multiple_of(w & 112, LANES)
                v = buf[r, pl.ds(coff, LANES)]
                rb = 2 * LANES * j
                rot_v[pl.ds(rb, LANES)] = v
                rot_v[pl.ds(rb + LANES, LANES)] = v
                t = rot_v[pl.ds(rb + (((w & 15) - j + LANES) & 15), LANES)]
                res = jnp.where(lane == j, res + t, res)
            out_v[pl.ds(goff, LANES)] = res
            return 0

        lax.fori_loop(0, CGP, gbody, 0)

        nxt = k + NSLOTS
        if nxt < NCH:
            handles[nxt] = start(nxt, slot)

    pltpu.sync_copy(out_v, out_hbm.at[pl.ds(base, B_PER_W)])


_SC_GATHER_ADD = functools.partial(
    pl.kernel,
    mesh=plsc.VectorSubcoreMesh(core_axis_name="c", subcore_axis_name="s"),
    out_type=jax.ShapeDtypeStruct((BATCH,), jnp.float32),
    scratch_types=[
        pltpu.VMEM((B_PER_W,), jnp.int32),
        pltpu.VMEM((B_PER_W,), jnp.int32),
        pltpu.VMEM((CH, 128), jnp.float32),
        pltpu.VMEM((CH, 128), jnp.float32),
        pltpu.VMEM((B_PER_W,), jnp.float32),
        pltpu.VMEM((2 * LANES * LANES,), jnp.float32),
        pltpu.VMEM((B_PER_W,), jnp.float32),
        pltpu.SemaphoreType.DMA,
        pltpu.SemaphoreType.DMA,
    ],
)(_sc_gather_body)


def kernel(user_ids, course_ids, user_factors, course_factors, fc_w, fc_b):
    nu = user_factors.shape[0]
    ncr = course_factors.shape[0]

    wu_row = fc_w[:N_FACTORS].T
    wc_row = fc_w[N_FACTORS:].T
    nru = -(-nu // 1024) * 8
    nrc = -(-ncr // 1024) * 8

    p_u = _tc_project(user_factors.T, wu_row, nru * 128)
    pu2 = p_u.reshape(nru, 128)
    bias = jnp.broadcast_to(fc_b, (BATCH,)).astype(jnp.float32)
    acc = _SC_GATHER_ADD(user_ids.astype(jnp.int32), pu2, bias)

    p_c = _tc_project(course_factors.T, wc_row, nrc * 128)
    pc2 = p_c.reshape(nrc, 128)
    return _SC_GATHER_ADD(course_ids.astype(jnp.int32), pc2, acc)
